# baseline (speedup 1.0000x reference)
import functools

import jax
import jax.lax as lax
import jax.numpy as jnp
from jax.experimental import pallas as pl
from jax.experimental.pallas import tpu as pltpu
from jax.experimental.pallas import tpu_sc as plsc

B, T, D = 4096, 200, 64
TD = T * D
NW = 32
GROUP = 8
GPW = TD // (GROUP * NW)
ROWS_W = GPW * GROUP

_mesh = plsc.VectorSubcoreMesh(core_axis_name="c", subcore_axis_name="s")


def _sc_body(x_hbm, t_hbm, o_hbm, tblv, buf, dsem, tsem):
    wid = lax.axis_index("s") * 2 + lax.axis_index("c")
    r0 = wid * ROWS_W
    pltpu.async_copy(t_hbm.at[pl.ds(r0, ROWS_W)], tblv, tsem).wait()

    def scale_body(i, _):
        sl = pl.ds(i * 16, 16)
        tblv[sl] = tblv[sl] * jnp.float32(D ** 0.5)
        return 0

    lax.fori_loop(0, ROWS_W // 16, scale_body, 0)

    def two_group_body(gg, _):
        v16 = tblv[pl.ds(gg * 16, 16)]
        for half in range(2):
            row = r0 + gg * 2 * GROUP + half * GROUP
            pltpu.async_copy(x_hbm.at[pl.ds(row, GROUP), :], buf, dsem).wait()
            vecs = [
                jnp.full((16,), v16[half * GROUP + r], jnp.float32)
                for r in range(GROUP)
            ]

            def cc_body(cc, _):
                sl = pl.ds(cc * 16, 16)
                for r in range(GROUP):
                    buf[r, sl] = buf[r, sl] + vecs[r]
                return 0

            lax.fori_loop(0, B // 16, cc_body, 0)
            pltpu.async_copy(buf, o_hbm.at[pl.ds(row, GROUP), :], dsem).wait()
        return 0

    lax.fori_loop(0, GPW // 2, two_group_body, 0)


_sc_call = functools.partial(
    pl.kernel,
    out_type=jax.ShapeDtypeStruct((TD, B), jnp.float32),
    mesh=_mesh,
    scratch_types=[
        pltpu.VMEM((ROWS_W,), jnp.float32),
        pltpu.VMEM((GROUP, B), jnp.float32),
        pltpu.SemaphoreType.DMA,
        pltpu.SemaphoreType.DMA,
    ],
)(_sc_body)


def kernel(inputs, lookup_table):
    scale_ignored = None
    del scale_ignored
    xt = jnp.transpose(inputs, (1, 2, 0)).reshape(TD, B)
    tbl = lookup_table.reshape(TD)
    out = _sc_call(xt, tbl)
    return jnp.transpose(out.reshape(T, D, B), (2, 0, 1))


# speedup vs baseline: 1.8681x; 1.8681x over previous
import functools

import jax
import jax.lax as lax
import jax.numpy as jnp
from jax.experimental import pallas as pl
from jax.experimental.pallas import tpu as pltpu
from jax.experimental.pallas import tpu_sc as plsc

B, T, D = 4096, 200, 64
TD = T * D
NW = 32
GROUP = 8
GPW = TD // (GROUP * NW)
ROWS_W = GPW * GROUP
SUPER = 6

_mesh = plsc.VectorSubcoreMesh(core_axis_name="c", subcore_axis_name="s")


def _sc_body(x_hbm, t_hbm, o_hbm, tblv, b0, b1, b2,
             is0, is1, is2, os0, os1, os2, tsem):
    bufs = (b0, b1, b2)
    isems = (is0, is1, is2)
    osems = (os0, os1, os2)
    wid = lax.axis_index("s") * 2 + lax.axis_index("c")
    r0 = wid * ROWS_W
    pltpu.async_copy(t_hbm.at[pl.ds(r0, ROWS_W)], tblv, tsem).wait()

    def scale_body(i, _):
        sl = pl.ds(i * 16, 16)
        tblv[sl] = tblv[sl] * jnp.float32(D ** 0.5)
        return 0

    lax.fori_loop(0, ROWS_W // 16, scale_body, 0)

    def start_in(p, row):
        pltpu.async_copy(x_hbm.at[pl.ds(row, GROUP), :], bufs[p], isems[p])

    def wait_in(p):
        pltpu.make_async_copy(
            x_hbm.at[pl.ds(0, GROUP), :], bufs[p], isems[p]).wait()

    def start_out(p, row):
        pltpu.async_copy(bufs[p], o_hbm.at[pl.ds(row, GROUP), :], osems[p])

    def wait_out(p):
        pltpu.make_async_copy(
            bufs[p], o_hbm.at[pl.ds(0, GROUP), :], osems[p]).wait()

    def compute(p, v16, half):
        buf = bufs[p]
        vecs = [
            jnp.full((16,), v16[half * GROUP + r], jnp.float32)
            for r in range(GROUP)
        ]

        def cc_body(cc, _):
            sl = pl.ds(cc * 16, 16)
            for r in range(GROUP):
                buf[r, sl] = buf[r, sl] + vecs[r]
            return 0

        lax.fori_loop(0, B // 16, cc_body, 0)

    start_in(0, r0)
    start_in(1, r0 + GROUP)

    def super_body(i, _):
        gbase = i * SUPER
        for j in range(SUPER):
            p = j % 3
            pm1 = (j + 2) % 3
            row = r0 + (gbase + j) * GROUP
            wait_in(p)
            if j % 2 == 0:
                v16 = tblv[pl.ds((gbase + j) * GROUP, 16)]
            compute(p, v16, j % 2)
            if j == 0:
                @pl.when(i > 0)
                def _():
                    wait_out(pm1)
            else:
                wait_out(pm1)
            start_out(p, row)
            start_in(pm1, row + 2 * GROUP)
        return 0

    nsuper = (GPW - 2) // SUPER
    lax.fori_loop(0, nsuper, super_body, 0)

    for g in range(nsuper * SUPER, GPW):
        p = g % 3
        row = r0 + g * GROUP
        wait_in(p)
        if g % 2 == 0:
            v16t = tblv[pl.ds(g * GROUP, 16)]
        compute(p, v16t, g % 2)
        if g >= 1:
            wait_out((g + 2) % 3)
        start_out(p, row)
        if g + 2 < GPW:
            start_in((g + 2) % 3, row + 2 * GROUP)
    wait_out((GPW - 1) % 3)


_sc_call = functools.partial(
    pl.kernel,
    out_type=jax.ShapeDtypeStruct((TD, B), jnp.float32),
    mesh=_mesh,
    scratch_types=[
        pltpu.VMEM((ROWS_W,), jnp.float32),
        pltpu.VMEM((GROUP, B), jnp.float32),
        pltpu.VMEM((GROUP, B), jnp.float32),
        pltpu.VMEM((GROUP, B), jnp.float32),
        pltpu.SemaphoreType.DMA,
        pltpu.SemaphoreType.DMA,
        pltpu.SemaphoreType.DMA,
        pltpu.SemaphoreType.DMA,
        pltpu.SemaphoreType.DMA,
        pltpu.SemaphoreType.DMA,
        pltpu.SemaphoreType.DMA,
    ],
)(_sc_body)


def kernel(inputs, lookup_table):
    xt = jnp.transpose(inputs, (1, 2, 0)).reshape(TD, B)
    tbl = lookup_table.reshape(TD)
    out = _sc_call(xt, tbl)
    return jnp.transpose(out.reshape(T, D, B), (2, 0, 1))


# speedup vs baseline: 2.0963x; 1.1222x over previous
import functools

import jax
import jax.lax as lax
import jax.numpy as jnp
from jax.experimental import pallas as pl
from jax.experimental.pallas import tpu as pltpu
from jax.experimental.pallas import tpu_sc as plsc

B, T, D = 4096, 200, 64
TD = T * D
NW = 32
GROUP = 8
GPW = TD // (GROUP * NW)
ROWS_W = GPW * GROUP
SUPER = 6

_mesh = plsc.VectorSubcoreMesh(core_axis_name="c", subcore_axis_name="s")


def _sc_body(x_hbm, t_hbm, o_hbm, tblv, b0, b1, b2,
             is0, is1, is2, os0, os1, os2, tsem):
    bufs = (b0, b1, b2)
    isems = (is0, is1, is2)
    osems = (os0, os1, os2)
    wid = lax.axis_index("s") * 2 + lax.axis_index("c")
    r0 = wid * ROWS_W
    pltpu.async_copy(t_hbm.at[pl.ds(r0, ROWS_W)], tblv, tsem).wait()

    def scale_body(i, _):
        sl = pl.ds(i * 16, 16)
        tblv[sl] = tblv[sl] * jnp.float32(D ** 0.5)
        return 0

    lax.fori_loop(0, ROWS_W // 16, scale_body, 0)

    def start_in(p, row):
        pltpu.async_copy(x_hbm.at[pl.ds(row, GROUP), :], bufs[p], isems[p])

    def wait_in(p):
        pltpu.make_async_copy(
            x_hbm.at[pl.ds(0, GROUP), :], bufs[p], isems[p]).wait()

    def start_out(p, row):
        pltpu.async_copy(bufs[p], o_hbm.at[pl.ds(row, GROUP), :], osems[p])

    def wait_out(p):
        pltpu.make_async_copy(
            bufs[p], o_hbm.at[pl.ds(0, GROUP), :], osems[p]).wait()

    def compute(p, v16, half):
        buf = bufs[p]
        vecs = [
            jnp.full((16,), v16[half * GROUP + r], jnp.float32)
            for r in range(GROUP)
        ]

        @plsc.parallel_loop(0, B // 16, 1, unroll=4)
        def cc_body(cc):
            sl = pl.ds(cc * 16, 16)
            for r in range(GROUP):
                buf[r, sl] = buf[r, sl] + vecs[r]

    start_in(0, r0)
    start_in(1, r0 + GROUP)

    def super_body(i, _):
        gbase = i * SUPER
        for j in range(SUPER):
            p = j % 3
            pm1 = (j + 2) % 3
            row = r0 + (gbase + j) * GROUP
            wait_in(p)
            if j % 2 == 0:
                v16 = tblv[pl.ds((gbase + j) * GROUP, 16)]
            compute(p, v16, j % 2)
            if j == 0:
                @pl.when(i > 0)
                def _():
                    wait_out(pm1)
            else:
                wait_out(pm1)
            start_out(p, row)
            start_in(pm1, row + 2 * GROUP)
        return 0

    nsuper = (GPW - 2) // SUPER
    lax.fori_loop(0, nsuper, super_body, 0)

    for g in range(nsuper * SUPER, GPW):
        p = g % 3
        row = r0 + g * GROUP
        wait_in(p)
        if g % 2 == 0:
            v16t = tblv[pl.ds(g * GROUP, 16)]
        compute(p, v16t, g % 2)
        if g >= 1:
            wait_out((g + 2) % 3)
        start_out(p, row)
        if g + 2 < GPW:
            start_in((g + 2) % 3, row + 2 * GROUP)
    wait_out((GPW - 1) % 3)


_sc_call = functools.partial(
    pl.kernel,
    out_type=jax.ShapeDtypeStruct((TD, B), jnp.float32),
    mesh=_mesh,
    scratch_types=[
        pltpu.VMEM((ROWS_W,), jnp.float32),
        pltpu.VMEM((GROUP, B), jnp.float32),
        pltpu.VMEM((GROUP, B), jnp.float32),
        pltpu.VMEM((GROUP, B), jnp.float32),
        pltpu.SemaphoreType.DMA,
        pltpu.SemaphoreType.DMA,
        pltpu.SemaphoreType.DMA,
        pltpu.SemaphoreType.DMA,
        pltpu.SemaphoreType.DMA,
        pltpu.SemaphoreType.DMA,
        pltpu.SemaphoreType.DMA,
    ],
)(_sc_body)


def kernel(inputs, lookup_table):
    xt = jnp.transpose(inputs, (1, 2, 0)).reshape(TD, B)
    tbl = lookup_table.reshape(TD)
    out = _sc_call(xt, tbl)
    return jnp.transpose(out.reshape(T, D, B), (2, 0, 1))


# speedup vs baseline: 2.1034x; 1.0034x over previous
import functools

import jax
import jax.lax as lax
import jax.numpy as jnp
from jax.experimental import pallas as pl
from jax.experimental.pallas import tpu as pltpu
from jax.experimental.pallas import tpu_sc as plsc

B, T, D = 4096, 200, 64
TD = T * D
NW = 32
GROUP = 8
GPW = TD // (GROUP * NW)
ROWS_W = GPW * GROUP
SUPER = 6

_mesh = plsc.VectorSubcoreMesh(core_axis_name="c", subcore_axis_name="s")


def _sc_body(x_hbm, t_hbm, o_hbm, tblv, b0, b1, b2,
             is0, is1, is2, os0, os1, os2, tsem):
    bufs = (b0, b1, b2)
    isems = (is0, is1, is2)
    osems = (os0, os1, os2)
    wid = lax.axis_index("s") * 2 + lax.axis_index("c")
    r0 = wid * ROWS_W
    pltpu.async_copy(t_hbm.at[pl.ds(r0, ROWS_W)], tblv, tsem).wait()

    def scale_body(i, _):
        sl = pl.ds(i * 16, 16)
        tblv[sl] = tblv[sl] * jnp.float32(D ** 0.5)
        return 0

    lax.fori_loop(0, ROWS_W // 16, scale_body, 0)

    def start_in(p, row):
        pltpu.async_copy(x_hbm.at[pl.ds(row, GROUP), :], bufs[p], isems[p])

    def wait_in(p):
        pltpu.make_async_copy(
            x_hbm.at[pl.ds(0, GROUP), :], bufs[p], isems[p]).wait()

    def start_out(p, row):
        pltpu.async_copy(bufs[p], o_hbm.at[pl.ds(row, GROUP), :], osems[p])

    def wait_out(p):
        pltpu.make_async_copy(
            bufs[p], o_hbm.at[pl.ds(0, GROUP), :], osems[p]).wait()

    def compute(p, v16, half):
        buf = bufs[p]
        vecs = [
            jnp.full((16,), v16[half * GROUP + r], jnp.float32)
            for r in range(GROUP)
        ]

        @plsc.parallel_loop(0, B // 16, 1, unroll=8)
        def cc_body(cc):
            sl = pl.ds(cc * 16, 16)
            for r in range(GROUP):
                buf[r, sl] = buf[r, sl] + vecs[r]

    start_in(0, r0)
    start_in(1, r0 + GROUP)

    def super_body(i, _):
        gbase = i * SUPER
        for j in range(SUPER):
            p = j % 3
            pm1 = (j + 2) % 3
            row = r0 + (gbase + j) * GROUP
            wait_in(p)
            if j % 2 == 0:
                v16 = tblv[pl.ds((gbase + j) * GROUP, 16)]
            compute(p, v16, j % 2)
            if j == 0:
                @pl.when(i > 0)
                def _():
                    wait_out(pm1)
            else:
                wait_out(pm1)
            start_out(p, row)
            start_in(pm1, row + 2 * GROUP)
        return 0

    nsuper = (GPW - 2) // SUPER
    lax.fori_loop(0, nsuper, super_body, 0)

    for g in range(nsuper * SUPER, GPW):
        p = g % 3
        row = r0 + g * GROUP
        wait_in(p)
        if g % 2 == 0:
            v16t = tblv[pl.ds(g * GROUP, 16)]
        compute(p, v16t, g % 2)
        if g >= 1:
            wait_out((g + 2) % 3)
        start_out(p, row)
        if g + 2 < GPW:
            start_in((g + 2) % 3, row + 2 * GROUP)
    wait_out((GPW - 1) % 3)


_sc_call = functools.partial(
    pl.kernel,
    out_type=jax.ShapeDtypeStruct((TD, B), jnp.float32),
    mesh=_mesh,
    scratch_types=[
        pltpu.VMEM((ROWS_W,), jnp.float32),
        pltpu.VMEM((GROUP, B), jnp.float32),
        pltpu.VMEM((GROUP, B), jnp.float32),
        pltpu.VMEM((GROUP, B), jnp.float32),
        pltpu.SemaphoreType.DMA,
        pltpu.SemaphoreType.DMA,
        pltpu.SemaphoreType.DMA,
        pltpu.SemaphoreType.DMA,
        pltpu.SemaphoreType.DMA,
        pltpu.SemaphoreType.DMA,
        pltpu.SemaphoreType.DMA,
    ],
)(_sc_body)


def kernel(inputs, lookup_table):
    xt = jnp.transpose(inputs, (1, 2, 0)).reshape(TD, B)
    tbl = lookup_table.reshape(TD)
    out = _sc_call(xt, tbl)
    return jnp.transpose(out.reshape(T, D, B), (2, 0, 1))
